# Initial kernel scaffold; baseline (speedup 1.0000x reference)
import jax
import jax.numpy as jnp
from jax.experimental import pallas as pl


def kernel(x, mask, W_q, W_k, W_v, W_o, W_router, W_gate, W1, W2, ln1_scale, ln1_bias, ln2_scale, ln2_bias):
    raise NotImplementedError("write your pallas kernel here")


# speedup vs baseline: 4.4122x; 4.4122x over previous
import functools
import jax
import jax.numpy as jnp
from jax import lax
from jax.experimental import pallas as pl
from jax.experimental.pallas import tpu as pltpu
from jax.experimental.pallas import tpu_sc as plsc

S = 2048
D = 768
H = 12
DH = 64
E = 8
DFF = 3072
CAP = 320
NLIST = E * CAP
NLIST_PAD = 2688
SBLK = 256
QBLK = 512

NC = 2
NS = 16
ROWS_PER_CORE = NLIST // NC
ROWS_PER_TILE = ROWS_PER_CORE // NS
TOK_PER_TILE = S // (NC * NS)


def _rowsum(x):
    n = x.shape[1]
    a = x[:, 0:128]
    for i in range(1, n // 128):
        a = a + x[:, 128 * i:128 * (i + 1)]
    r = a[:, 0:8]
    for v in range(1, 16):
        r = r + a[:, 8 * v:8 * v + 8]
    for sh in (4, 2, 1):
        r = r[:, 0:sh] + r[:, sh:2 * sh]
    return r


def _ln_rows(x, scale, bias):
    n = x.shape[1]
    m = _rowsum(x) * (1.0 / n)
    v = _rowsum((x - m) ** 2) * (1.0 / n)
    return (x - m) / jnp.sqrt(v + 1e-5) * scale + bias


def _dot_t(a, b):
    return lax.dot_general(a.astype(jnp.bfloat16), b.astype(jnp.bfloat16),
                           (((1,), (1,)), ((), ())),
                           preferred_element_type=jnp.float32)


def _dot_bf(a, b):
    return jnp.dot(a.astype(jnp.bfloat16), b.astype(jnp.bfloat16),
                   preferred_element_type=jnp.float32)


def _prep_body(x_ref, s1_ref, b1_ref, wre_ref, xn_ref, idx_ref, cnt_ref):
    i = pl.program_id(0)
    xn = _ln_rows(x_ref[...], s1_ref[...], b1_ref[...])
    xn_ref[...] = xn
    gl = _dot_t(xn, wre_ref[...])
    best = gl[:, 0:H]
    bidx = jnp.zeros((SBLK, H), jnp.int32)
    for e in range(1, E):
        cand = gl[:, e * H:(e + 1) * H]
        upd = cand > best
        best = jnp.where(upd, cand, best)
        bidx = jnp.where(upd, e, bidx)
    idx_ref[...] = bidx

    @pl.when(i == 0)
    def _():
        cnt_ref[...] = jnp.zeros((E, H), jnp.float32)

    rows = []
    for e in range(E):
        rows.append(jnp.sum((bidx == e).astype(jnp.float32), axis=0,
                            keepdims=True))
    cnt_ref[...] += jnp.concatenate(rows, axis=0)


def _prep_call(x, ln1_scale, ln1_bias, w_re):
    return pl.pallas_call(
        _prep_body,
        grid=(S // SBLK,),
        in_specs=[
            pl.BlockSpec((SBLK, D), lambda i: (i, 0)),
            pl.BlockSpec((1, D), lambda i: (0, 0)),
            pl.BlockSpec((1, D), lambda i: (0, 0)),
            pl.BlockSpec((E * H, D), lambda i: (0, 0)),
        ],
        out_specs=[
            pl.BlockSpec((SBLK, D), lambda i: (i, 0)),
            pl.BlockSpec((SBLK, H), lambda i: (i, 0)),
            pl.BlockSpec((E, H), lambda i: (0, 0)),
        ],
        out_shape=[
            jax.ShapeDtypeStruct((S, D), jnp.float32),
            jax.ShapeDtypeStruct((S, H), jnp.int32),
            jax.ShapeDtypeStruct((E, H), jnp.float32),
        ],
    )(x, ln1_scale, ln1_bias, w_re)


def _heads_body(xn_ref, xh_ref, wq_ref, wk_ref, wv_ref, idx_ref,
                q_ref, k_ref, v_ref):
    h = pl.program_id(0)
    xn = xn_ref[...]
    q_ref[0] = _dot_t(xn, wq_ref[0])
    k_ref[0] = _dot_t(xn, wk_ref[0])
    iota = lax.broadcasted_iota(jnp.int32, (S, H), 1)
    col = jnp.sum(jnp.where(iota == h, idx_ref[...], 0), axis=1,
                  keepdims=True)
    xh = xh_ref[0]
    acc = jnp.zeros((S, DH), jnp.float32)
    for e in range(E):
        ve = _dot_bf(xh, wv_ref[0, e])
        acc = acc + jnp.where(col == e, ve, 0.0)
    v_ref[0] = acc


def _heads_call(xn, xh3, wq3, wk3, w_v, idx_a):
    return pl.pallas_call(
        _heads_body,
        grid=(H,),
        in_specs=[
            pl.BlockSpec((S, D), lambda h: (0, 0)),
            pl.BlockSpec((1, S, DH), lambda h: (h, 0, 0)),
            pl.BlockSpec((1, DH, D), lambda h: (h, 0, 0)),
            pl.BlockSpec((1, DH, D), lambda h: (h, 0, 0)),
            pl.BlockSpec((1, E, DH, DH), lambda h: (h, 0, 0, 0)),
            pl.BlockSpec((S, H), lambda h: (0, 0)),
        ],
        out_specs=[
            pl.BlockSpec((1, S, DH), lambda h: (h, 0, 0)),
            pl.BlockSpec((1, S, DH), lambda h: (h, 0, 0)),
            pl.BlockSpec((1, S, DH), lambda h: (h, 0, 0)),
        ],
        out_shape=[
            jax.ShapeDtypeStruct((H, S, DH), jnp.float32),
            jax.ShapeDtypeStruct((H, S, DH), jnp.float32),
            jax.ShapeDtypeStruct((H, S, DH), jnp.float32),
        ],
    )(xn, xh3, wq3, wk3, w_v, idx_a)


def _attn_body(q_ref, k_ref, v_ref, m_ref, wo_ref, idx_ref, y_ref, acc_ref):
    h = pl.program_id(0)
    qb = pl.program_id(1)
    sc = _dot_t(q_ref[0], k_ref[0])
    sc = sc * 0.125 + m_ref[pl.ds(qb * QBLK, QBLK), :]
    mx = jnp.max(sc, axis=1, keepdims=True)
    p = jnp.exp(sc - mx)
    p = p / _rowsum(p)
    pv = lax.dot_general(p.astype(jnp.bfloat16), v_ref[0].astype(jnp.bfloat16),
                         (((0,), (0,)), ((), ())),
                         preferred_element_type=jnp.float32)

    @pl.when(qb == 0)
    def _():
        acc_ref[...] = jnp.zeros((S, DH), jnp.float32)

    acc_ref[...] += pv

    @pl.when(qb == S // QBLK - 1)
    def _():
        attn = acc_ref[...]
        iota = lax.broadcasted_iota(jnp.int32, (S, H), 1)
        col = jnp.sum(jnp.where(iota == h, idx_ref[...], 0), axis=1,
                      keepdims=True)
        acc = jnp.zeros((S, DH), jnp.float32)
        for e in range(E):
            oe = _dot_bf(attn, wo_ref[0, e])
            acc = acc + jnp.where(col == e, oe, 0.0)
        y_ref[0] = acc


def _attn_call(q3, k3, v3, mask, w_o, idx_a):
    return pl.pallas_call(
        _attn_body,
        grid=(H, S // QBLK),
        in_specs=[
            pl.BlockSpec((1, QBLK, DH), lambda h, qb: (h, qb, 0)),
            pl.BlockSpec((1, S, DH), lambda h, qb: (h, 0, 0)),
            pl.BlockSpec((1, QBLK, DH), lambda h, qb: (h, qb, 0)),
            pl.BlockSpec((S, S), lambda h, qb: (0, 0)),
            pl.BlockSpec((1, E, DH, DH), lambda h, qb: (h, 0, 0, 0)),
            pl.BlockSpec((S, H), lambda h, qb: (0, 0)),
        ],
        out_specs=pl.BlockSpec((1, S, DH), lambda h, qb: (h, 0, 0)),
        out_shape=jax.ShapeDtypeStruct((H, S, DH), jnp.float32),
        scratch_shapes=[pltpu.VMEM((S, DH), jnp.float32)],
    )(q3, k3, v3, mask, w_o, idx_a)


def _route_body(x_ref, y_ref, s2_ref, b2_ref, wg_ref, cnta_ref,
                h1_ref, hn_ref, pd1_ref, pd2_ref, k1_ref, k2_ref,
                w1_ref, w2_ref, loss_ref):
    h1 = x_ref[...] + y_ref[...]
    h1_ref[...] = h1
    hn = _ln_rows(h1, s2_ref[...], b2_ref[...])
    hn_ref[...] = hn
    g = _dot_t(hn, wg_ref[...])
    iota = lax.broadcasted_iota(jnp.int32, (S, E), 1)
    v1 = jnp.max(g, axis=1, keepdims=True)
    i1 = jnp.min(jnp.where(g == v1, iota, E), axis=1, keepdims=True)
    gm = jnp.where(iota == i1, -jnp.inf, g)
    v2 = jnp.max(gm, axis=1, keepdims=True)
    i2 = jnp.min(jnp.where(gm == v2, iota, E), axis=1, keepdims=True)
    t = jnp.exp(v2 - v1)
    p1 = 1.0 / (1.0 + t)
    p2 = t / (1.0 + t)

    c8 = ((iota == i1) | (iota == i2)).astype(jnp.float32)
    cc = c8
    sh = 1
    while sh < S:
        top = jnp.zeros((sh, E), jnp.float32)
        cc = cc + jnp.concatenate([top, cc[:S - sh, :]], axis=0)
        sh *= 2
    texc = cc - c8
    r1 = jnp.sum(jnp.where(iota == i1, texc, 0.0), axis=1,
                 keepdims=True) + 1.0
    r2 = jnp.sum(jnp.where(iota == i2, texc, 0.0), axis=1,
                 keepdims=True) + 1.0
    keep1 = r1 <= float(CAP)
    keep2 = r2 <= float(CAP)
    pos1 = i1 * CAP + r1.astype(jnp.int32) - 1
    pos2 = i2 * CAP + r2.astype(jnp.int32) - 1
    pd1_ref[...] = jnp.where(keep1, pos1, 0)
    pd2_ref[...] = jnp.where(keep2, pos2, 0)
    k1_ref[...] = keep1.astype(jnp.int32)
    k2_ref[...] = keep2.astype(jnp.int32)
    k1f = keep1.astype(jnp.float32)
    k2f = keep2.astype(jnp.float32)
    a1 = p1 * k1f
    a2 = p2 * k2f
    den = a1 + a2 + 1e-9
    w1 = a1 / den
    w2 = a2 / den
    w1_ref[...] = w1
    w2_ref[...] = w2

    c1 = k1f * (p1 > 0.0).astype(jnp.float32)
    c2 = k2f * (p2 > 0.0).astype(jnp.float32)
    cnt = jnp.sum(((iota == i1).astype(jnp.float32) * c1 +
                   (iota == i2).astype(jnp.float32) * c2), axis=0,
                  keepdims=True)
    total = jnp.sum(cnt)
    pa = cnt / (total + 1e-9)
    loss_f = jnp.where(total > 0.0, jnp.sum(pa * pa) * E, 0.0)

    ema = 0.01 * cnta_ref[...] / float(S)
    lb = ema / (jnp.sum(ema) + 1e-9)
    loss_a = jnp.sum(lb * lb) * (E * H)
    loss_ref[...] = jnp.full((1, 1), 0.0) + loss_a + loss_f


def _route_call(x, y, ln2_scale, ln2_bias, w_gate, cnt_a):
    full = lambda shape: pl.BlockSpec(shape, lambda: tuple(0 for _ in shape))
    return pl.pallas_call(
        _route_body,
        in_specs=[
            full((S, D)), full((S, D)), full((1, D)), full((1, D)),
            full((E, D)), full((E, H)),
        ],
        out_specs=[
            full((S, D)), full((S, D)),
            full((S, 1)), full((S, 1)), full((S, 1)), full((S, 1)),
            full((S, 1)), full((S, 1)), full((1, 1)),
        ],
        out_shape=[
            jax.ShapeDtypeStruct((S, D), jnp.float32),
            jax.ShapeDtypeStruct((S, D), jnp.float32),
            jax.ShapeDtypeStruct((S, 1), jnp.int32),
            jax.ShapeDtypeStruct((S, 1), jnp.int32),
            jax.ShapeDtypeStruct((S, 1), jnp.int32),
            jax.ShapeDtypeStruct((S, 1), jnp.int32),
            jax.ShapeDtypeStruct((S, 1), jnp.float32),
            jax.ShapeDtypeStruct((S, 1), jnp.float32),
            jax.ShapeDtypeStruct((1, 1), jnp.float32),
        ],
    )(x, y, ln2_scale, ln2_bias, w_gate, cnt_a)


def _sc_dispatch_body(hn_hbm, pd1_hbm, pd2_hbm, k1_hbm, k2_hbm, xg_hbm,
                      list_sh, pd1_v, pd2_v, k1_v, k2_v, list_v, idx_v,
                      rows_v, sem):
    c = lax.axis_index("c")
    s = lax.axis_index("s")

    @pl.when(s == 0)
    def _():
        pltpu.sync_copy(pd1_hbm, pd1_v)
        pltpu.sync_copy(pd2_hbm, pd2_v)
        pltpu.sync_copy(k1_hbm, k1_v)
        pltpu.sync_copy(k2_hbm, k2_v)

        def zero_body(i, carry):
            list_v[pl.ds(i * 16, 16)] = jnp.zeros((16,), jnp.int32)
            return carry

        lax.fori_loop(0, NLIST_PAD // 16, zero_body, 0)

        def scat_body(i, carry):
            tok = i * 16 + lax.iota(jnp.int32, 16)
            p1 = pd1_v[pl.ds(i * 16, 16)]
            m1 = k1_v[pl.ds(i * 16, 16)] > 0
            plsc.store_scatter(list_v, [p1], tok, mask=m1)
            p2 = pd2_v[pl.ds(i * 16, 16)]
            m2 = k2_v[pl.ds(i * 16, 16)] > 0
            plsc.store_scatter(list_v, [p2], tok, mask=m2)
            return carry

        lax.fori_loop(0, S // 16, scat_body, 0)
        pltpu.sync_copy(list_v, list_sh)

    plsc.subcore_barrier()
    base = c * ROWS_PER_CORE + s * ROWS_PER_TILE
    pltpu.sync_copy(list_sh.at[pl.ds(base, ROWS_PER_TILE)], idx_v)
    pltpu.async_copy(hn_hbm.at[idx_v], rows_v, sem).wait()
    pltpu.sync_copy(rows_v, xg_hbm.at[pl.ds(base, ROWS_PER_TILE)])


def _sc_dispatch_call(hn, pd1, pd2, k1, k2):
    mesh = plsc.VectorSubcoreMesh(core_axis_name="c", subcore_axis_name="s")
    fn = functools.partial(
        pl.kernel,
        out_type=jax.ShapeDtypeStruct((NLIST, D), jnp.float32),
        mesh=mesh,
        compiler_params=pltpu.CompilerParams(needs_layout_passes=False),
        scratch_types=[
            pltpu.VMEM_SHARED((NLIST_PAD,), jnp.int32),
            pltpu.VMEM((S,), jnp.int32),
            pltpu.VMEM((S,), jnp.int32),
            pltpu.VMEM((S,), jnp.int32),
            pltpu.VMEM((S,), jnp.int32),
            pltpu.VMEM((NLIST_PAD,), jnp.int32),
            pltpu.VMEM((ROWS_PER_TILE,), jnp.int32),
            pltpu.VMEM((ROWS_PER_TILE, D), jnp.float32),
            pltpu.SemaphoreType.DMA,
        ],
    )(_sc_dispatch_body)
    return fn(hn, pd1, pd2, k1, k2)


def _ffn_body(xg_ref, w1_ref, w2_ref, y_ref):
    j = pl.program_id(1)
    xb = xg_ref[...].astype(jnp.bfloat16)
    hb = jax.nn.relu(
        jnp.dot(xb, w1_ref[0].astype(jnp.bfloat16),
                preferred_element_type=jnp.float32)).astype(jnp.bfloat16)
    yp = jnp.dot(hb, w2_ref[0].astype(jnp.bfloat16),
                 preferred_element_type=jnp.float32)

    @pl.when(j == 0)
    def _():
        y_ref[...] = yp

    @pl.when(j == 1)
    def _():
        y_ref[...] += yp


def _ffn_call(xg, w1, w2):
    dhalf = DFF // 2
    return pl.pallas_call(
        _ffn_body,
        grid=(E, 2),
        in_specs=[
            pl.BlockSpec((CAP, D), lambda e, j: (e, 0)),
            pl.BlockSpec((1, D, dhalf), lambda e, j: (e, 0, j)),
            pl.BlockSpec((1, dhalf, D), lambda e, j: (e, j, 0)),
        ],
        out_specs=pl.BlockSpec((CAP, D), lambda e, j: (e, 0)),
        out_shape=jax.ShapeDtypeStruct((NLIST, D), jnp.float32),
    )(xg, w1, w2)


def _sc_combine_body(y_hbm, pc1_hbm, pc2_hbm, g1_hbm, g2_hbm,
                     idx_v, rows_v, sem):
    c = lax.axis_index("c")
    s = lax.axis_index("s")
    base = (c * NS + s) * TOK_PER_TILE
    pltpu.sync_copy(pc1_hbm.at[pl.ds(base, TOK_PER_TILE)], idx_v)
    pltpu.async_copy(y_hbm.at[idx_v], rows_v, sem).wait()
    pltpu.sync_copy(rows_v, g1_hbm.at[pl.ds(base, TOK_PER_TILE)])
    pltpu.sync_copy(pc2_hbm.at[pl.ds(base, TOK_PER_TILE)], idx_v)
    pltpu.async_copy(y_hbm.at[idx_v], rows_v, sem).wait()
    pltpu.sync_copy(rows_v, g2_hbm.at[pl.ds(base, TOK_PER_TILE)])


def _sc_combine_call(y, pc1, pc2):
    mesh = plsc.VectorSubcoreMesh(core_axis_name="c", subcore_axis_name="s")
    fn = functools.partial(
        pl.kernel,
        out_type=[
            jax.ShapeDtypeStruct((S, D), jnp.float32),
            jax.ShapeDtypeStruct((S, D), jnp.float32),
        ],
        mesh=mesh,
        scratch_types=[
            pltpu.VMEM((TOK_PER_TILE,), jnp.int32),
            pltpu.VMEM((TOK_PER_TILE, D), jnp.float32),
            pltpu.SemaphoreType.DMA,
        ],
    )(_sc_combine_body)
    return fn(y, pc1, pc2)


def _combine_body(h1_ref, g1_ref, g2_ref, w1_ref, w2_ref, out_ref):
    out_ref[...] = (h1_ref[...] + w1_ref[...] * g1_ref[...] +
                    w2_ref[...] * g2_ref[...])


def _combine_call(h1, g1, g2, w1, w2):
    return pl.pallas_call(
        _combine_body,
        grid=(S // SBLK,),
        in_specs=[
            pl.BlockSpec((SBLK, D), lambda i: (i, 0)),
            pl.BlockSpec((SBLK, D), lambda i: (i, 0)),
            pl.BlockSpec((SBLK, D), lambda i: (i, 0)),
            pl.BlockSpec((SBLK, 1), lambda i: (i, 0)),
            pl.BlockSpec((SBLK, 1), lambda i: (i, 0)),
        ],
        out_specs=pl.BlockSpec((SBLK, D), lambda i: (i, 0)),
        out_shape=jax.ShapeDtypeStruct((S, D), jnp.float32),
    )(h1, g1, g2, w1, w2)


def kernel(x, mask, W_q, W_k, W_v, W_o, W_router, W_gate, W1, W2,
           ln1_scale, ln1_bias, ln2_scale, ln2_bias):
    x2 = x.reshape(S, D)
    s1 = ln1_scale.reshape(1, D)
    b1 = ln1_bias.reshape(1, D)
    s2 = ln2_scale.reshape(1, D)
    b2 = ln2_bias.reshape(1, D)
    w_re = W_router.reshape(H, E, D).transpose(1, 0, 2).reshape(E * H, D)
    wq3 = W_q.reshape(H, DH, D)
    wk3 = W_k.reshape(H, DH, D)

    xn, idx_a, cnt_a = _prep_call(x2, s1, b1, w_re)
    xh3 = xn.reshape(S, H, DH).transpose(1, 0, 2)
    q3, k3, v3 = _heads_call(xn, xh3, wq3, wk3, W_v, idx_a)
    y3 = _attn_call(q3, k3, v3, mask, W_o, idx_a)
    y = y3.transpose(1, 0, 2).reshape(S, D)

    (h1, hn, pd1, pd2, kk1, kk2, w1, w2, loss) = _route_call(
        x2, y, s2, b2, W_gate, cnt_a)

    xg = _sc_dispatch_call(hn, pd1.reshape(S), pd2.reshape(S),
                           kk1.reshape(S), kk2.reshape(S))
    yexp = _ffn_call(xg, W1, W2)
    g1, g2 = _sc_combine_call(yexp, pd1.reshape(S), pd2.reshape(S))
    out = _combine_call(h1, g1, g2, w1, w2)
    return out.reshape(1, S, D), loss[0, 0]
